# Initial kernel scaffold; baseline (speedup 1.0000x reference)
import jax
import jax.numpy as jnp
from jax.experimental import pallas as pl


def kernel(image_feature, start_predictions, h0, state_transform, image_ids, embed, W_h, W_f, W_out):
    raise NotImplementedError("write your pallas kernel here")


# speedup vs baseline: 43.4494x; 43.4494x over previous
import jax
import jax.numpy as jnp
from jax.experimental import pallas as pl
from jax.experimental.pallas import tpu as pltpu

_B = 16
_S = 4
_BEAM = 5
_PNB = 5
_V = 1000
_VP = 1024
_H = 1024
_T = 12
_END = 2
_NEG = -1e20
_J = _S * _BEAM
_JP = 24


def _iota(shape, dim):
    return jax.lax.broadcasted_iota(jnp.int32, shape, dim)


def _top5_lastdim(x):
    iv = _iota(x.shape, 2)
    vals, idxs = [], []
    for _ in range(_PNB):
        m = jnp.max(x, axis=2, keepdims=True)
        ix = jnp.min(jnp.where(x == m, iv, x.shape[2]), axis=2, keepdims=True)
        vals.append(m)
        idxs.append(ix)
        x = jnp.where(iv == ix, -jnp.inf, x)
    return vals, idxs


def _beam_kernel(feat_ref, sp_ref, h0_ref, mask0_ref, masks_ref,
                 embed_ref, wh_ref, wf_ref, wout_ref, pred_out, lp_out):
    f32 = jnp.float32
    feat = feat_ref[...]
    embed = embed_ref[...]
    wh = wh_ref[...]
    wout = wout_ref[...]

    fw = jnp.dot(feat, wf_ref[...], preferred_element_type=f32)

    iv2 = _iota((_B, _VP), 1)
    oh0 = (sp_ref[...] == iv2).astype(f32)
    z = (jnp.dot(h0_ref[...], wh, preferred_element_type=f32)
         + jnp.dot(oh0, embed, preferred_element_type=f32, precision=jax.lax.Precision.HIGHEST) + fw)
    h1 = jnp.tanh(z)
    logits = jnp.dot(h1, wout, preferred_element_type=f32)
    logits = jnp.where(iv2 < _V, logits, -1e30)
    m0 = jnp.max(logits, axis=1, keepdims=True)
    lp0 = logits - m0 - jnp.log(jnp.sum(jnp.exp(logits - m0), axis=1,
                                        keepdims=True))

    pred_cols, lp_cols = [], []
    for s in range(_S):
        x = jnp.where(mask0_ref[s] != 0, lp0, _NEG)
        x = jnp.where(iv2 < _V, x, -jnp.inf)
        for _ in range(_BEAM):
            mm = jnp.max(x, axis=1, keepdims=True)
            ix = jnp.min(jnp.where(x == mm, iv2, _VP), axis=1, keepdims=True)
            pred_cols.append(ix)
            lp_cols.append(mm)
            x = jnp.where(iv2 == ix, -jnp.inf, x)
    pad_i = jnp.zeros((_B, _JP - _J), jnp.int32)
    pad_f = jnp.full((_B, _JP - _J), -jnp.inf, f32)
    preds = jnp.concatenate(pred_cols + [pad_i], axis=1)
    last_lp = jnp.concatenate(lp_cols + [pad_f], axis=1)

    pred_seq = [preds]
    bp_seq = []

    h3 = jnp.broadcast_to(h1[:, None, :], (_B, _JP, _H))
    fw2 = jnp.broadcast_to(fw[:, None, :], (_B, _JP, _H)).reshape(_B * _JP, _H)
    iv3 = _iota((_B, _JP, _VP), 2)
    ij2 = _iota((_B, _JP), 1)
    ik3 = _iota((_B, _JP, _PNB), 2)
    ij3p = _iota((_B, _JP, _PNB), 1)
    ae3 = jnp.where(iv3 == _END, 0.0, _NEG)

    for _ in range(_T - 1):
        oh3 = (preds[:, :, None] == iv3).astype(f32)
        z2 = (jnp.dot(h3.reshape(_B * _JP, _H), wh, preferred_element_type=f32)
              + jnp.dot(oh3.reshape(_B * _JP, _VP), embed,
                        preferred_element_type=f32,
                        precision=jax.lax.Precision.HIGHEST) + fw2)
        hn2 = jnp.tanh(z2)
        lg3 = jnp.dot(hn2, wout, preferred_element_type=f32).reshape(_B, _JP, _VP)
        lg3 = jnp.where(iv3 < _V, lg3, -1e30)
        m1 = jnp.max(lg3, axis=2, keepdims=True)
        lp3 = lg3 - m1 - jnp.log(jnp.sum(jnp.exp(lg3 - m1), axis=2,
                                         keepdims=True))
        cleaned = jnp.where(preds[:, :, None] == _END, ae3, lp3)
        hn3 = hn2.reshape(_B, _JP, _H)

        np_cols, nl_cols, nb_cols = [], [], []
        for i in range(_S):
            xs = jnp.where(masks_ref[i] != 0, cleaned, _NEG)
            vals, idxs = _top5_lastdim(xs)
            cand_lp = jnp.concatenate(vals, axis=2) + last_lp[:, :, None]
            cand_pc = jnp.concatenate(idxs, axis=2)
            x3 = cand_lp
            for _r in range(_BEAM):
                mj = jnp.max(x3, axis=2)
                kj = jnp.min(jnp.where(x3 == mj[:, :, None], ik3, _PNB),
                             axis=2)
                mb = jnp.max(mj, axis=1, keepdims=True)
                js = jnp.min(jnp.where(mj == mb, ij2, _JP), axis=1,
                             keepdims=True)
                sel_j = ij2 == js
                ks = jnp.sum(jnp.where(sel_j, kj, 0), axis=1, keepdims=True)
                pc_k = jnp.sum(jnp.where(ik3 == kj[:, :, None], cand_pc, 0),
                               axis=2)
                tok = jnp.sum(jnp.where(sel_j, pc_k, 0), axis=1, keepdims=True)
                nl_cols.append(mb)
                nb_cols.append(js)
                np_cols.append(tok)
                x3 = jnp.where((ij3p == js[:, :, None])
                               & (ik3 == ks[:, :, None]), -jnp.inf, x3)
        preds = jnp.concatenate(np_cols + [pad_i], axis=1)
        last_lp = jnp.concatenate(nl_cols + [pad_f], axis=1)
        bp = jnp.concatenate(nb_cols + [pad_i], axis=1)
        pred_seq.append(preds)
        bp_seq.append(bp)

        new_h = hn3
        for k in range(_J):
            new_h = jnp.where(bp[:, :, None] == k, hn3[:, k:k + 1, :], new_h)
        h3 = new_h

    def gather20(xx, idx):
        out = xx
        for k in range(_J):
            out = jnp.where(idx == k, xx[:, k:k + 1], out)
        return out

    seq = [None] * _T
    seq[_T - 1] = pred_seq[_T - 1]
    cur = bp_seq[_T - 2]
    for t in range(_T - 2, 0, -1):
        seq[t] = gather20(pred_seq[t], cur)
        cur = gather20(bp_seq[t - 1], cur)
    seq[0] = gather20(pred_seq[0], cur)

    for t in range(_T):
        pred_out[t] = seq[t]
    lp_out[...] = last_lp


def _call(interpret=False):
    return pl.pallas_call(
        _beam_kernel,
        out_shape=(jax.ShapeDtypeStruct((_T, _B, _JP), jnp.int32),
                   jax.ShapeDtypeStruct((_B, _JP), jnp.float32)),
        compiler_params=pltpu.CompilerParams(
            vmem_limit_bytes=128 * 1024 * 1024),
        interpret=interpret,
    )


def kernel(image_feature, start_predictions, h0, state_transform, image_ids,
           embed, W_h, W_f, W_out):
    sp2 = start_predictions.astype(jnp.int32).reshape(_B, 1)
    embed_p = jnp.zeros((_VP, _H), jnp.float32).at[:_V].set(embed)
    W_out_p = jnp.zeros((_H, _VP), jnp.float32).at[:, :_V].set(W_out)
    st = state_transform.astype(jnp.int8)
    mask0 = jnp.zeros((_S, _B, _VP), jnp.int8).at[:, :, :_V].set(
        jnp.transpose(st[:, 0], (1, 0, 2)))
    stt = jnp.transpose(st, (2, 0, 1, 3))
    stt = jnp.broadcast_to(stt[:, :, :, None, :],
                           (_S, _B, _S, _BEAM, _V)).reshape(_S, _B, _J, _V)
    masks = jnp.zeros((_S, _B, _JP, _VP), jnp.int8).at[:, :, :_J, :_V].set(stt)
    pred_out, lp = _call()(image_feature, sp2, h0, mask0, masks,
                           embed_p, W_h, W_f, W_out_p)
    all_pred = jnp.transpose(pred_out[:, :, :_J], (1, 2, 0)).reshape(
        _B, _S, _BEAM, _T)
    return all_pred, lp[:, :_J].reshape(_B, _S, _BEAM)
